# Initial kernel scaffold; baseline (speedup 1.0000x reference)
import functools

import jax
import jax.numpy as jnp
from jax import lax
from jax.experimental import pallas as pl
from jax.experimental.pallas import tpu as pltpu
from jax.experimental.pallas import tpu_sc as plsc

N = 10000
E = 320000
NNZ = 320000
D = 128

NC = 2
NS = 16
NW = NC * NS
EPW = E // NW
CH = 128
NCHUNK = (EPW + CH - 1) // CH
PW = NCHUNK * CH
WORDS_PER_TILE = N * D // NS
ZROWS = 125

_mesh = plsc.VectorSubcoreMesh(core_axis_name="c", subcore_axis_name="s")


def _zero_fill(zbuf):
    zero16 = jnp.zeros((16,), jnp.float32)

    def body(i, _):
        for q in range(D // 16):
            zbuf[i, pl.ds(q * 16, 16)] = zero16
        return 0

    lax.fori_loop(0, ZROWS, body, 0)


def _densify_body(xr, xc, xv, out, acc, st_r, st_c, st_v, idx2, zbuf):
    c = lax.axis_index("c")
    s = lax.axis_index("s")
    wid = c * NS + s

    _zero_fill(zbuf)
    zflat = zbuf.reshape(ZROWS * D)
    for q in range(WORDS_PER_TILE // (ZROWS * D)):
        pltpu.sync_copy(
            zflat, acc.at[pl.ds(s * WORDS_PER_TILE + q * ZROWS * D, ZROWS * D)])
    plsc.subcore_barrier()

    pltpu.sync_copy(xr.at[wid], st_r)
    pltpu.sync_copy(xc.at[wid], st_c)
    pltpu.sync_copy(xv.at[wid], st_v)

    def idx_body(j, _):
        for q in range(CH // 16):
            r = st_r[j, pl.ds(q * 16, 16)]
            cc = st_c[j, pl.ds(q * 16, 16)]
            idx2[j, pl.ds(q * 16, 16)] = r * D + cc
        return 0

    lax.fori_loop(0, NCHUNK, idx_body, 0)

    def scat_body(j, _):
        pltpu.sync_copy(st_v.at[j], acc.at[idx2.at[j]], add=True)
        return 0

    lax.fori_loop(0, NCHUNK, scat_body, 0)
    plsc.subcore_barrier()

    pltpu.sync_copy(acc.at[pl.ds(s * WORDS_PER_TILE, WORDS_PER_TILE)],
                    out.at[c, pl.ds(s * WORDS_PER_TILE, WORDS_PER_TILE)])


_densify = functools.partial(
    pl.kernel,
    out_type=jax.ShapeDtypeStruct((NC, N * D), jnp.float32),
    mesh=_mesh,
    scratch_types=[
        pltpu.VMEM_SHARED((N * D,), jnp.float32),
        pltpu.VMEM((NCHUNK, CH), jnp.int32),
        pltpu.VMEM((NCHUNK, CH), jnp.int32),
        pltpu.VMEM((NCHUNK, CH), jnp.float32),
        pltpu.VMEM((NCHUNK, CH), jnp.int32),
        pltpu.VMEM((ZROWS, D), jnp.float32),
    ],
)(_densify_body)


def _propagate_body(xw, ar, ac, av, out, acc, st_r, st_c, st_v, rbuf, zbuf,
                    sem):
    c = lax.axis_index("c")
    s = lax.axis_index("s")
    wid = c * NS + s

    _zero_fill(zbuf)
    rows_per_tile = N // NS
    for q in range(rows_per_tile // ZROWS):
        pltpu.sync_copy(zbuf,
                        acc.at[pl.ds(s * rows_per_tile + q * ZROWS, ZROWS)])
    plsc.subcore_barrier()

    pltpu.sync_copy(ar.at[wid], st_r)
    pltpu.sync_copy(ac.at[wid], st_c)
    pltpu.sync_copy(av.at[wid], st_v)

    def chunk_body(j, _):
        pltpu.async_copy(xw.at[st_c.at[j]], rbuf, sem).wait()

        def row_body(i, _):
            w = st_v[j, i]
            for q in range(D // 16):
                rbuf[i, pl.ds(q * 16, 16)] = rbuf[i, pl.ds(q * 16, 16)] * w
            return 0

        lax.fori_loop(0, CH, row_body, 0)

        pltpu.sync_copy(rbuf, acc.at[st_r.at[j]], add=True)
        return 0

    lax.fori_loop(0, NCHUNK, chunk_body, 0)
    plsc.subcore_barrier()

    pltpu.sync_copy(acc.at[pl.ds(s * rows_per_tile, rows_per_tile)],
                    out.at[c, pl.ds(s * rows_per_tile, rows_per_tile)])


_propagate = functools.partial(
    pl.kernel,
    out_type=jax.ShapeDtypeStruct((NC, N, D), jnp.float32),
    mesh=_mesh,
    scratch_types=[
        pltpu.VMEM_SHARED((N, D), jnp.float32),
        pltpu.VMEM((NCHUNK, CH), jnp.int32),
        pltpu.VMEM((NCHUNK, CH), jnp.int32),
        pltpu.VMEM((NCHUNK, CH), jnp.float32),
        pltpu.VMEM((CH, D), jnp.float32),
        pltpu.VMEM((ZROWS, D), jnp.float32),
        pltpu.SemaphoreType.DMA,
    ],
)(_propagate_body)


def _matmul_body(xd_ref, w_ref, o_ref):
    o_ref[...] = jnp.dot(xd_ref[0] + xd_ref[1], w_ref[...],
                         preferred_element_type=jnp.float32)


def _finish_body(a_ref, b_ref, o_ref):
    o_ref[...] = jnp.maximum(a_ref[0] + a_ref[1] + b_ref[...], 0.0)


_BLK = 400


def _pad_split(a, fill):
    a = a.reshape(NW, EPW)
    a = jnp.pad(a, ((0, 0), (0, PW - EPW)), constant_values=fill)
    return a.reshape(NW, NCHUNK, CH)


def kernel(x_rows, x_cols, x_vals, adj_rows, adj_cols, adj_vals, W, b):
    xr = _pad_split(x_rows, 0)
    xc = _pad_split(x_cols, 0)
    xv = _pad_split(x_vals, 0.0)
    ar = _pad_split(adj_rows, 0)
    ac = _pad_split(adj_cols, 0)
    av = _pad_split(adj_vals, 0.0)

    xd = _densify(xr, xc, xv).reshape(NC, N, D)

    xw = pl.pallas_call(
        _matmul_body,
        grid=(N // _BLK,),
        in_specs=[
            pl.BlockSpec((NC, _BLK, D), lambda i: (0, i, 0)),
            pl.BlockSpec((D, D), lambda i: (0, 0)),
        ],
        out_specs=pl.BlockSpec((_BLK, D), lambda i: (i, 0)),
        out_shape=jax.ShapeDtypeStruct((N, D), jnp.float32),
    )(xd, W)

    ax = _propagate(xw, ar, ac, av)

    out = pl.pallas_call(
        _finish_body,
        grid=(N // _BLK,),
        in_specs=[
            pl.BlockSpec((NC, _BLK, D), lambda i: (0, i, 0)),
            pl.BlockSpec((1, D), lambda i: (0, 0)),
        ],
        out_specs=pl.BlockSpec((_BLK, D), lambda i: (i, 0)),
        out_shape=jax.ShapeDtypeStruct((N, D), jnp.float32),
    )(ax, b.reshape(1, D))

    return out


# speedup vs baseline: 7.6704x; 7.6704x over previous
import functools

import jax
import jax.numpy as jnp
from jax import lax
from jax.experimental import pallas as pl
from jax.experimental.pallas import tpu as pltpu
from jax.experimental.pallas import tpu_sc as plsc

N = 10000
E = 320000
NNZ = 320000
D = 128

NC = 2
NS = 16
NW = NC * NS
EPW = E // NW
CH = 128
NCHUNK = (EPW + CH - 1) // CH
PW = NCHUNK * CH
WORDS_PER_TILE = N * D // NS
ZWORDS = 4000
N_PAD = 10240
RPT = N_PAD // NS

_mesh = plsc.VectorSubcoreMesh(core_axis_name="c", subcore_axis_name="s")


def _zero_fill_rows(zbuf, nrows):
    zero16 = jnp.zeros((16,), jnp.float32)

    def body(i, _):
        for q in range(D // 16):
            zbuf[i, pl.ds(q * 16, 16)] = zero16
        return 0

    lax.fori_loop(0, nrows, body, 0)


def _zero_fill_flat(zbuf, nwords):
    zero16 = jnp.zeros((16,), jnp.float32)

    def body(i, _):
        zbuf[pl.ds(i * 16, 16)] = zero16
        return 0

    lax.fori_loop(0, nwords // 16, body, 0)


def _densify_body(xr, xc, xv, out, acc, st_r, st_c, st_v, idx2, zbuf):
    c = lax.axis_index("c")
    s = lax.axis_index("s")
    wid = c * NS + s

    _zero_fill_flat(zbuf, ZWORDS)
    for q in range(WORDS_PER_TILE // ZWORDS):
        pltpu.sync_copy(
            zbuf, acc.at[pl.ds(s * WORDS_PER_TILE + q * ZWORDS, ZWORDS)])
    plsc.subcore_barrier()

    pltpu.sync_copy(xr.at[wid], st_r)
    pltpu.sync_copy(xc.at[wid], st_c)
    pltpu.sync_copy(xv.at[wid], st_v)

    def idx_body(j, _):
        for q in range(CH // 16):
            r = st_r[j, pl.ds(q * 16, 16)]
            cc = st_c[j, pl.ds(q * 16, 16)]
            idx2[j, pl.ds(q * 16, 16)] = r * D + cc
        return 0

    lax.fori_loop(0, NCHUNK, idx_body, 0)

    def scat_body(j, _):
        pltpu.sync_copy(st_v.at[j], acc.at[idx2.at[j]], add=True)
        return 0

    lax.fori_loop(0, NCHUNK, scat_body, 0)
    plsc.subcore_barrier()

    pltpu.sync_copy(acc.at[pl.ds(s * WORDS_PER_TILE, WORDS_PER_TILE)],
                    out.at[pl.ds(wid * WORDS_PER_TILE, WORDS_PER_TILE)])


_densify = functools.partial(
    pl.kernel,
    out_type=jax.ShapeDtypeStruct((NC * N * D,), jnp.float32),
    mesh=_mesh,
    scratch_types=[
        pltpu.VMEM_SHARED((N * D,), jnp.float32),
        pltpu.VMEM((NCHUNK, CH), jnp.int32),
        pltpu.VMEM((NCHUNK, CH), jnp.int32),
        pltpu.VMEM((NCHUNK, CH), jnp.float32),
        pltpu.VMEM((NCHUNK, CH), jnp.int32),
        pltpu.VMEM((ZWORDS,), jnp.float32),
    ],
)(_densify_body)


def _propagate_body(xw, ar, ac, av, out, acc, st_r, st_c, st_v, rbuf, sem):
    c = lax.axis_index("c")
    s = lax.axis_index("s")
    wid = c * NS + s

    _zero_fill_rows(rbuf, CH)
    for q in range(RPT // CH):
        pltpu.sync_copy(rbuf, acc.at[pl.ds(s * RPT + q * CH, CH)])
    plsc.subcore_barrier()

    pltpu.sync_copy(ar.at[wid], st_r)
    pltpu.sync_copy(ac.at[wid], st_c)
    pltpu.sync_copy(av.at[wid], st_v)

    def chunk_body(j, _):
        pltpu.async_copy(xw.at[st_c.at[j]], rbuf, sem).wait()

        def row_grp(b, _):
            v = st_v[j, pl.ds(b * 16, 16)]
            for i in range(16):
                w = v[i]
                for q in range(D // 16):
                    rbuf[b * 16 + i, pl.ds(q * 16, 16)] = (
                        rbuf[b * 16 + i, pl.ds(q * 16, 16)] * w)
            return 0

        lax.fori_loop(0, CH // 16, row_grp, 0)

        pltpu.sync_copy(rbuf, acc.at[st_r.at[j]], add=True)
        return 0

    lax.fori_loop(0, NCHUNK, chunk_body, 0)
    plsc.subcore_barrier()

    pltpu.sync_copy(acc.at[pl.ds(s * RPT, RPT)],
                    out.at[c, pl.ds(s * RPT, RPT)])


_propagate = functools.partial(
    pl.kernel,
    out_type=jax.ShapeDtypeStruct((NC, N_PAD, D), jnp.float32),
    mesh=_mesh,
    scratch_types=[
        pltpu.VMEM_SHARED((N_PAD, D), jnp.float32),
        pltpu.VMEM((NCHUNK, CH), jnp.int32),
        pltpu.VMEM((NCHUNK, CH), jnp.int32),
        pltpu.VMEM((NCHUNK, CH), jnp.float32),
        pltpu.VMEM((CH, D), jnp.float32),
        pltpu.SemaphoreType.DMA,
    ],
)(_propagate_body)


def _matmul_body(xd_ref, w_ref, o_ref):
    o_ref[...] = jnp.dot(xd_ref[0] + xd_ref[1], w_ref[...],
                         preferred_element_type=jnp.float32)


def _finish_body(a_ref, b_ref, o_ref):
    o_ref[...] = jnp.maximum(a_ref[0] + a_ref[1] + b_ref[...], 0.0)


_BLK = 400


def _pad_split(a, fill):
    a = a.reshape(NW, EPW)
    a = jnp.pad(a, ((0, 0), (0, PW - EPW)), constant_values=fill)
    return a.reshape(NW, NCHUNK, CH)


def kernel(x_rows, x_cols, x_vals, adj_rows, adj_cols, adj_vals, W, b):
    xr = _pad_split(x_rows, 0)
    xc = _pad_split(x_cols, 0)
    xv = _pad_split(x_vals, 0.0)
    ar = _pad_split(adj_rows, 0)
    ac = _pad_split(adj_cols, 0)
    av = _pad_split(adj_vals, 0.0)

    xd = _densify(xr, xc, xv).reshape(NC, N, D)

    xw = pl.pallas_call(
        _matmul_body,
        grid=(N // _BLK,),
        in_specs=[
            pl.BlockSpec((NC, _BLK, D), lambda i: (0, i, 0)),
            pl.BlockSpec((D, D), lambda i: (0, 0)),
        ],
        out_specs=pl.BlockSpec((_BLK, D), lambda i: (i, 0)),
        out_shape=jax.ShapeDtypeStruct((N, D), jnp.float32),
    )(xd, W)

    ax = _propagate(xw, ar, ac, av)

    out = pl.pallas_call(
        _finish_body,
        grid=(N // _BLK,),
        in_specs=[
            pl.BlockSpec((NC, _BLK, D), lambda i: (0, i, 0)),
            pl.BlockSpec((1, D), lambda i: (0, 0)),
        ],
        out_specs=pl.BlockSpec((_BLK, D), lambda i: (i, 0)),
        out_shape=jax.ShapeDtypeStruct((N, D), jnp.float32),
    )(ax, b.reshape(1, D))

    return out
